# baseline (speedup 1.0000x reference)
import jax
import jax.numpy as jnp
from jax import lax
from jax.experimental import pallas as pl
from jax.experimental.pallas import tpu as pltpu
from jax.experimental.pallas import tpu_sc as plsc

_N_ATOMS = 819200
_N_FEATURES = 64
_TABLE_ROWS = 83
_NC = 2
_NS = 16
_NUM_WORKERS = _NC * _NS
_B_PER_W = _N_ATOMS // _NUM_WORKERS
_RPG = 128
_GPC = 2
_CHUNK = _RPG * _GPC
_N_CHUNKS = _B_PER_W // _CHUNK
_IDX_ROWS = _B_PER_W // _RPG


def _emb_body(x_hbm, table_hbm, out_hbm, table_sh, win_sh, idx_v,
              rows0, rows1, gsem0, gsem1, ssem0, ssem1, dsem):
    cid = lax.axis_index("c")
    sid = lax.axis_index("s")
    wid = sid * _NC + cid
    row_base = wid * _B_PER_W

    @pl.when(sid == 0)
    def _stage_table():
        pltpu.sync_copy(table_hbm, rows0.at[pl.ds(0, _TABLE_ROWS)])
        pltpu.sync_copy(rows0.at[pl.ds(0, _TABLE_ROWS)], table_sh)

    pltpu.sync_copy(x_hbm.at[pl.ds(wid * _IDX_ROWS, _IDX_ROWS)], idx_v)
    plsc.subcore_barrier()

    rows = (rows0, rows1)
    gsem = (gsem0, gsem1)
    ssem = (ssem0, ssem1)

    def gathers(i, b):
        for k in range(_GPC):
            pltpu.async_copy(
                table_sh.at[idx_v.at[i * _GPC + k]],
                rows[b].at[pl.ds(k * _RPG, _RPG)],
                gsem[b])
        pltpu.make_async_copy(out_hbm.at[pl.ds(0, _CHUNK)], rows[b],
                              gsem[b]).wait()

    def spill(i, b):
        pltpu.async_copy(rows[b], win_sh.at[b, sid], ssem[b])
        pltpu.make_async_copy(rows[b], win_sh.at[b, sid], ssem[b]).wait()

    def fire_dma(i, v):
        for s2 in range(_NS):
            dst0 = (s2 * _NC + cid) * _B_PER_W + i * _CHUNK
            pltpu.async_copy(win_sh.at[v, s2],
                             out_hbm.at[pl.ds(dst0, _CHUNK)], dsem)

    def wait_dma(v):
        for s2 in range(_NS):
            pltpu.make_async_copy(win_sh.at[v, s2],
                                  out_hbm.at[pl.ds(0, _CHUNK)], dsem).wait()

    for b in range(2):
        gathers(b, b)
        spill(b, b)
        plsc.subcore_barrier()

        @pl.when(sid == 0)
        def _dma0(b=b):
            fire_dma(b, b)

    def pair(j, carry):
        for b in range(2):
            i = 2 * j + b

            @pl.when(sid == 0)
            def _drain(b=b):
                wait_dma(b)

            plsc.subcore_barrier()
            gathers(i, b)
            spill(i, b)
            plsc.subcore_barrier()

            @pl.when(sid == 0)
            def _dma(i=i, b=b):
                fire_dma(i, b)
        return carry

    lax.fori_loop(1, _N_CHUNKS // 2, pair, 0)

    @pl.when(sid == 0)
    def _drain_tail():
        wait_dma(0)
        wait_dma(1)

    plsc.subcore_barrier()


@jax.jit
def kernel(x, embedding_list):
    run = pl.kernel(
        _emb_body,
        out_type=jax.ShapeDtypeStruct((_N_ATOMS, _N_FEATURES), jnp.float32),
        mesh=plsc.VectorSubcoreMesh(core_axis_name="c", subcore_axis_name="s"),
        scratch_types=[
            pltpu.VMEM_SHARED((_TABLE_ROWS, _N_FEATURES), jnp.float32),
            pltpu.VMEM_SHARED((2, _NS, _CHUNK, _N_FEATURES), jnp.float32),
            pltpu.VMEM((_IDX_ROWS, _RPG), jnp.int32),
            pltpu.VMEM((_CHUNK, _N_FEATURES), jnp.float32),
            pltpu.VMEM((_CHUNK, _N_FEATURES), jnp.float32),
            pltpu.SemaphoreType.DMA,
            pltpu.SemaphoreType.DMA,
            pltpu.SemaphoreType.DMA,
            pltpu.SemaphoreType.DMA,
            pltpu.SemaphoreType.DMA,
        ],
        compiler_params=pltpu.CompilerParams(use_tc_tiling_on_sc=False,
                                             needs_layout_passes=False),
    )
    return run(x.reshape(-1, _RPG), embedding_list)


# speedup vs baseline: 1.0139x; 1.0139x over previous
import jax
import jax.numpy as jnp
from jax import lax
from jax.experimental import pallas as pl
from jax.experimental.pallas import tpu as pltpu
from jax.experimental.pallas import tpu_sc as plsc

_N_ATOMS = 819200
_N_FEATURES = 64
_TABLE_ROWS = 83
_NC = 2
_NS = 16
_NUM_WORKERS = _NC * _NS
_B_PER_W = _N_ATOMS // _NUM_WORKERS
_RPG = 128
_GPC = 2
_CHUNK = _RPG * _GPC
_N_CHUNKS = _B_PER_W // _CHUNK
_IDX_ROWS = _B_PER_W // _RPG


def _emb_body(x_hbm, table_hbm, out_hbm, table_sh, win_sh, idx_v,
              rows0, rows1, gsem0, gsem1, ssem0, ssem1, dsem):
    cid = lax.axis_index("c")
    sid = lax.axis_index("s")
    wid = sid * _NC + cid

    @pl.when(sid == 0)
    def _stage_table():
        pltpu.sync_copy(table_hbm, rows0.at[pl.ds(0, _TABLE_ROWS)])
        pltpu.sync_copy(rows0.at[pl.ds(0, _TABLE_ROWS)], table_sh)

    pltpu.sync_copy(x_hbm.at[pl.ds(wid * _IDX_ROWS, _IDX_ROWS)], idx_v)
    plsc.subcore_barrier()

    rows = (rows0, rows1)
    gsem = (gsem0, gsem1)
    ssem = (ssem0, ssem1)

    def fire_gathers(i, b):
        for k in range(_GPC):
            pltpu.async_copy(
                table_sh.at[idx_v.at[i * _GPC + k]],
                rows[b].at[pl.ds(k * _RPG, _RPG)],
                gsem[b])

    def wait_gathers(b):
        pltpu.make_async_copy(out_hbm.at[pl.ds(0, _CHUNK)], rows[b],
                              gsem[b]).wait()

    def fire_spill(b):
        pltpu.async_copy(rows[b], win_sh.at[b, sid], ssem[b])

    def wait_spill(b):
        pltpu.make_async_copy(rows[b], win_sh.at[b, sid], ssem[b]).wait()

    def fire_dma(i, v):
        for s2 in range(_NS):
            dst0 = (s2 * _NC + cid) * _B_PER_W + i * _CHUNK
            pltpu.async_copy(win_sh.at[v, s2],
                             out_hbm.at[pl.ds(dst0, _CHUNK)], dsem)

    def wait_dma(v):
        for s2 in range(_NS):
            pltpu.make_async_copy(win_sh.at[v, s2],
                                  out_hbm.at[pl.ds(0, _CHUNK)], dsem).wait()

    def step(i, b, first, fire_next):
        wait_gathers(b)
        if not first:
            @pl.when(sid == 0)
            def _drain():
                wait_dma(b)
        plsc.subcore_barrier()
        fire_spill(b)
        if fire_next:
            fire_gathers(i + 1, 1 - b)
        wait_spill(b)
        plsc.subcore_barrier()

        @pl.when(sid == 0)
        def _push():
            fire_dma(i, b)

    fire_gathers(0, 0)
    step(0, 0, True, True)
    step(1, 1, True, True)

    def pair(j, carry):
        for b in range(2):
            step(2 * j + b, b, False, True)
        return carry

    lax.fori_loop(1, _N_CHUNKS // 2 - 1, pair, 0)

    step(_N_CHUNKS - 2, 0, False, True)
    step(_N_CHUNKS - 1, 1, False, False)

    @pl.when(sid == 0)
    def _drain_tail():
        wait_dma(0)
        wait_dma(1)

    plsc.subcore_barrier()


@jax.jit
def kernel(x, embedding_list):
    run = pl.kernel(
        _emb_body,
        out_type=jax.ShapeDtypeStruct((_N_ATOMS, _N_FEATURES), jnp.float32),
        mesh=plsc.VectorSubcoreMesh(core_axis_name="c", subcore_axis_name="s"),
        scratch_types=[
            pltpu.VMEM_SHARED((_TABLE_ROWS, _N_FEATURES), jnp.float32),
            pltpu.VMEM_SHARED((2, _NS, _CHUNK, _N_FEATURES), jnp.float32),
            pltpu.VMEM((_IDX_ROWS, _RPG), jnp.int32),
            pltpu.VMEM((_CHUNK, _N_FEATURES), jnp.float32),
            pltpu.VMEM((_CHUNK, _N_FEATURES), jnp.float32),
            pltpu.SemaphoreType.DMA,
            pltpu.SemaphoreType.DMA,
            pltpu.SemaphoreType.DMA,
            pltpu.SemaphoreType.DMA,
            pltpu.SemaphoreType.DMA,
        ],
        compiler_params=pltpu.CompilerParams(use_tc_tiling_on_sc=False,
                                             needs_layout_passes=False),
    )
    return run(x.reshape(-1, _RPG), embedding_list)
